# baseline (speedup 1.0000x reference)
import functools

import jax
import jax.numpy as jnp
from jax import lax
from jax.experimental import pallas as pl
from jax.experimental.pallas import tpu as pltpu
from jax.experimental.pallas import tpu_sc as plsc

D_MODEL = 64
NBUF = 8


@jax.jit
def _sc_embedding_lookup(x, table):
    n_seq, seq_len = x.shape
    info = plsc.get_sparse_core_info()
    nc, ns = info.num_cores, info.num_subcores
    nw = nc * ns
    seq_per_w = n_seq // nw
    assert seq_per_w % NBUF == 0

    mesh = plsc.VectorSubcoreMesh(core_axis_name="c", subcore_axis_name="s")

    @functools.partial(
        pl.kernel,
        mesh=mesh,
        out_type=jax.ShapeDtypeStruct((n_seq, seq_len, D_MODEL), jnp.float32),
        scratch_types=[
            pltpu.VMEM((seq_per_w, seq_len), jnp.int32),
            pltpu.VMEM((NBUF, seq_len, D_MODEL), jnp.float32),
            [pltpu.SemaphoreType.DMA] * NBUF,
        ],
        compiler_params=pltpu.CompilerParams(use_tc_tiling_on_sc=False),
    )
    def k(x_hbm, table_hbm, out_hbm, idx_v, rows_v, sems):
        wid = lax.axis_index("s") * nc + lax.axis_index("c")
        seq_base = wid * seq_per_w
        pltpu.sync_copy(x_hbm.at[pl.ds(seq_base, seq_per_w)], idx_v)

        def gather(j, b):
            pltpu.async_copy(table_hbm.at[idx_v.at[j]], rows_v.at[b], sems[b])

        def wait_gather(j, b):
            pltpu.make_async_copy(
                table_hbm.at[idx_v.at[j]], rows_v.at[b], sems[b]
            ).wait()

        def write_out(j, b):
            pltpu.sync_copy(rows_v.at[b], out_hbm.at[seq_base + j])

        for b in range(NBUF):
            gather(b, b)

        def ring_body(t, carry):
            j0 = t * NBUF
            for b in range(NBUF):
                j = j0 + b
                wait_gather(j, b)
                write_out(j, b)
                gather(j + NBUF, b)
            return carry

        lax.fori_loop(0, seq_per_w // NBUF - 1, ring_body, 0)

        j0 = seq_per_w - NBUF
        for b in range(NBUF):
            j = j0 + b
            wait_gather(j, b)
            write_out(j, b)

    return k(x, table)


def kernel(x, table):
    return _sc_embedding_lookup(x, table)


# speedup vs baseline: 1.0890x; 1.0890x over previous
import functools

import jax
import jax.numpy as jnp
from jax import lax
from jax.experimental import pallas as pl
from jax.experimental.pallas import tpu as pltpu
from jax.experimental.pallas import tpu_sc as plsc

D_MODEL = 64
NBUF = 8


@jax.jit
def _sc_embedding_lookup(x2, table_padded_rows):
    n_seq, seq_len = x2.shape
    info = plsc.get_sparse_core_info()
    nc, ns = info.num_cores, info.num_subcores
    nw = nc * ns
    seq_per_w = n_seq // nw
    assert seq_per_w % NBUF == 0

    mesh = plsc.VectorSubcoreMesh(core_axis_name="c", subcore_axis_name="s")

    @functools.partial(
        pl.kernel,
        mesh=mesh,
        out_type=jax.ShapeDtypeStruct((n_seq, seq_len, D_MODEL), jnp.float32),
        scratch_types=[
            pltpu.VMEM((seq_per_w, seq_len), jnp.int32),
            pltpu.VMEM((NBUF, seq_len, D_MODEL), jnp.float32),
            [pltpu.SemaphoreType.DMA] * NBUF,
        ],
        compiler_params=pltpu.CompilerParams(use_tc_tiling_on_sc=False),
    )
    def k(x_hbm, table_hbm, out_hbm, idx_v, rows_v, sems):
        wid = lax.axis_index("s") * nc + lax.axis_index("c")
        seq_base = wid * seq_per_w
        pltpu.sync_copy(x_hbm.at[pl.ds(seq_base, seq_per_w)], idx_v)

        def gather(j, b):
            pltpu.async_copy(table_hbm.at[idx_v.at[j]], rows_v.at[b], sems[b])

        def wait_gather(j, b):
            pltpu.make_async_copy(
                table_hbm.at[idx_v.at[j]], rows_v.at[b], sems[b]
            ).wait()

        def write_out(j, b):
            pltpu.sync_copy(rows_v.at[b], out_hbm.at[seq_base + j])

        for b in range(NBUF):
            gather(b, b)

        def ring_body(t, carry):
            j0 = t * NBUF
            for b in range(NBUF):
                j = j0 + b
                wait_gather(j, b)
                write_out(j, b)
                gather(j + NBUF, b)
            return carry

        lax.fori_loop(0, seq_per_w // NBUF - 1, ring_body, 0)

        j0 = seq_per_w - NBUF
        for b in range(NBUF):
            j = j0 + b
            wait_gather(j, b)
            write_out(j, b)

    return k(x2, table_padded_rows)


def kernel(x, table):
    tbl = jnp.pad(table, ((0, 0), (0, 64))).reshape(2 * table.shape[0], D_MODEL)
    return _sc_embedding_lookup(x * 2, tbl)


# speedup vs baseline: 1.2014x; 1.1032x over previous
import functools

import jax
import jax.numpy as jnp
from jax import lax
from jax.experimental import pallas as pl
from jax.experimental.pallas import tpu as pltpu
from jax.experimental.pallas import tpu_sc as plsc

D_MODEL = 64
NBUF = 8
T_BLOCK = 2048


def _transpose_block(tt_ref, out_ref):
    out_ref[:, 0:64] = tt_ref[...].T


@jax.jit
def _tc_pad_transpose(tt):
    d, v = tt.shape
    grid = (v + T_BLOCK - 1) // T_BLOCK
    return pl.pallas_call(
        _transpose_block,
        grid=(grid,),
        in_specs=[pl.BlockSpec((d, T_BLOCK), lambda i: (0, i))],
        out_specs=pl.BlockSpec((T_BLOCK, 2 * d), lambda i: (i, 0)),
        out_shape=jax.ShapeDtypeStruct((v, 2 * d), jnp.float32),
        compiler_params=pltpu.CompilerParams(
            dimension_semantics=("arbitrary",),
        ),
    )(tt)


@jax.jit
def _sc_embedding_lookup(x2, table_padded_rows):
    n_seq, seq_len = x2.shape
    info = plsc.get_sparse_core_info()
    nc, ns = info.num_cores, info.num_subcores
    nw = nc * ns
    seq_per_w = n_seq // nw
    assert seq_per_w % NBUF == 0

    mesh = plsc.VectorSubcoreMesh(core_axis_name="c", subcore_axis_name="s")

    @functools.partial(
        pl.kernel,
        mesh=mesh,
        out_type=jax.ShapeDtypeStruct((n_seq, seq_len, D_MODEL), jnp.float32),
        scratch_types=[
            pltpu.VMEM((seq_per_w, seq_len), jnp.int32),
            pltpu.VMEM((NBUF, seq_len, D_MODEL), jnp.float32),
            [pltpu.SemaphoreType.DMA] * NBUF,
        ],
        compiler_params=pltpu.CompilerParams(use_tc_tiling_on_sc=False),
    )
    def k(x_hbm, table_hbm, out_hbm, idx_v, rows_v, sems):
        wid = lax.axis_index("s") * nc + lax.axis_index("c")
        seq_base = wid * seq_per_w
        pltpu.sync_copy(x_hbm.at[pl.ds(seq_base, seq_per_w)], idx_v)

        def gather(j, b):
            pltpu.async_copy(table_hbm.at[idx_v.at[j]], rows_v.at[b], sems[b])

        def wait_gather(j, b):
            pltpu.make_async_copy(
                table_hbm.at[idx_v.at[j]], rows_v.at[b], sems[b]
            ).wait()

        def write_out(j, b):
            pltpu.sync_copy(rows_v.at[b], out_hbm.at[seq_base + j])

        for b in range(NBUF):
            gather(b, b)

        def ring_body(t, carry):
            j0 = t * NBUF
            for b in range(NBUF):
                j = j0 + b
                wait_gather(j, b)
                write_out(j, b)
                gather(j + NBUF, b)
            return carry

        lax.fori_loop(0, seq_per_w // NBUF - 1, ring_body, 0)

        j0 = seq_per_w - NBUF
        for b in range(NBUF):
            j = j0 + b
            wait_gather(j, b)
            write_out(j, b)

    return k(x2, table_padded_rows)


def kernel(x, table):
    tbl128 = _tc_pad_transpose(table.T)
    tbl = tbl128.reshape(2 * table.shape[0], D_MODEL)
    return _sc_embedding_lookup(x * 2, tbl)
